# baseline (speedup 1.0000x reference)
import functools
import math

import jax
import jax.numpy as jnp
import numpy as np
from jax import lax
from jax.experimental import pallas as pl
from jax.experimental.pallas import tpu as pltpu
from jax.experimental.pallas import tpu_sc as plsc

VOCAB = 100000
D_MODEL = 128
MAX_LEN = 50
BATCH = 4096
SEQ = 50

N_TOKENS = BATCH * SEQ
CHUNK = 200
GATHER = 50
NBUF = 4
SCALE = math.sqrt(D_MODEL)


def _make_pe():
    pe = np.zeros((MAX_LEN, D_MODEL), dtype=np.float32)
    position = np.arange(MAX_LEN, dtype=np.float32)[:, None]
    div_term = np.exp(
        np.arange(0, D_MODEL, 2, dtype=np.float32) * -(math.log(10000.0) / D_MODEL)
    )
    pe[:, 0::2] = np.sin(position * div_term)
    pe[:, 1::2] = np.cos(position * div_term)
    return pe


_PE = _make_pe()


def kernel(encoded_words, embed_table):
    info = plsc.get_sparse_core_info()
    nw = info.num_cores * info.num_subcores
    b_per_w = N_TOKENS // nw
    n_chunks = b_per_w // CHUNK

    n_g = CHUNK // GATHER
    idx = encoded_words.astype(jnp.int32).reshape(nw, n_chunks, n_g, GATHER)
    pe_in = jnp.asarray(_PE)

    mesh = plsc.VectorSubcoreMesh(core_axis_name="c", subcore_axis_name="s")

    @functools.partial(
        pl.kernel,
        mesh=mesh,
        out_type=jax.ShapeDtypeStruct((N_TOKENS, D_MODEL), jnp.float32),
        scratch_types=[
            pltpu.VMEM((n_chunks, n_g, GATHER), jnp.int32),
            pltpu.VMEM((MAX_LEN, D_MODEL), jnp.float32),
        ]
        + [pltpu.VMEM((CHUNK, D_MODEL), jnp.float32)] * NBUF
        + [pltpu.SemaphoreType.DMA] * (2 * NBUF),
    )
    def run(table_hbm, idx_hbm, pe_hbm, out_hbm, idx_v, pe_v, *bufs_sems):
        bufs = bufs_sems[:NBUF]
        gsems = bufs_sems[NBUF:2 * NBUF]
        ssems = bufs_sems[2 * NBUF:]
        wid = lax.axis_index("s") * info.num_cores + lax.axis_index("c")
        base = wid * b_per_w
        pltpu.sync_copy(idx_hbm.at[wid], idx_v)
        pltpu.sync_copy(pe_hbm, pe_v)

        def gather_start(j, p):
            for g in range(n_g):
                pltpu.async_copy(
                    table_hbm.at[idx_v.at[j, g]],
                    bufs[p].at[pl.ds(g * GATHER, GATHER)],
                    gsems[p],
                )

        def gather_wait(j, p):
            for g in range(n_g):
                pltpu.make_async_copy(
                    table_hbm.at[idx_v.at[j, g]],
                    bufs[p].at[pl.ds(g * GATHER, GATHER)],
                    gsems[p],
                ).wait()

        def store_start(j, p):
            pltpu.async_copy(
                bufs[p], out_hbm.at[pl.ds(base + j * CHUNK, CHUNK)], ssems[p]
            )

        def store_wait(j, p):
            pltpu.make_async_copy(
                bufs[p], out_hbm.at[pl.ds(base + j * CHUNK, CHUNK)], ssems[p]
            ).wait()

        def compute(p):
            buf = bufs[p]

            def row_body(r, _):
                pe_regs = [pe_v[r, pl.ds(c * 16, 16)] for c in range(D_MODEL // 16)]
                for s in range(CHUNK // MAX_LEN):
                    off = s * MAX_LEN
                    for c in range(D_MODEL // 16):
                        sl = pl.ds(c * 16, 16)
                        buf[off + r, sl] = buf[off + r, sl] * SCALE + pe_regs[c]
                return 0

            lax.fori_loop(0, MAX_LEN, row_body, 0)

        gather_start(0, 0)
        gather_start(1, 1)

        def quad_body(jj, _):
            for p in range(NBUF):
                j = jj * NBUF + p
                gather_wait(j, p)
                q = (p + 2) % NBUF

                @pl.when(j + 2 < n_chunks)
                def _():
                    @pl.when(j >= 2)
                    def _():
                        store_wait(j - 2, q)

                    gather_start(j + 2, q)

                compute(p)
                store_start(j, p)

            return 0

        lax.fori_loop(0, n_chunks // NBUF, quad_body, 0)
        for p in range(NBUF):
            store_wait(n_chunks - NBUF + p, p)

    out = run(embed_table, idx, pe_in)
    return out.reshape(BATCH, SEQ, D_MODEL)
